# initial kernel scaffold (unmeasured)
import jax
import jax.numpy as jnp
from jax import lax
from jax.experimental import pallas as pl
from jax.experimental.pallas import tpu as pltpu

N_Z = 4
SCALE = 64 ** -0.5


def kernel(Q, K, V):
    B, Sq, H, D = Q.shape
    _, Kloc, _, _ = K.shape
    HD = H * D
    LW = HD + H

    K2 = K.reshape(B, Kloc, HD)
    V2 = V.reshape(B, Kloc, HD)

    eye = jnp.eye(H, dtype=Q.dtype)
    QB = (Q[:, 0, :, :, None] * eye[:, None, :]).reshape(B, HD, H)

    M = (jnp.arange(HD)[None, :] // D == jnp.arange(H)[:, None]).astype(
        jnp.float32
    )

    def body(qb_ref, k_ref, v_ref, m_ref, out_ref, acc_ref, c0_ref, c1_ref,
             send_sems, recv_sems):
        b = pl.program_id(0)

        s = lax.dot_general(
            k_ref[0], qb_ref[0],
            (((1,), (0,)), ((), ())),
            preferred_element_type=jnp.float32,
        )
        p = jnp.exp(s * SCALE)
        l = jnp.sum(p, axis=0, keepdims=True)
        o2 = lax.dot_general(
            p, v_ref[0],
            (((0,), (0,)), ((), ())),
            preferred_element_type=jnp.float32,
        )
        o_flat = jnp.sum(o2 * m_ref[:, :], axis=0, keepdims=True)

        acc_ref[pl.ds(b, 1), :HD] = o_flat
        acc_ref[pl.ds(b, 1), HD:] = l

        @pl.when(b == B - 1)
        def _comm():
            xi = lax.axis_index("x")
            yi = lax.axis_index("y")
            zi = lax.axis_index("z")
            p0 = zi + 1 - 2 * lax.rem(zi, 2)
            p1 = lax.rem(zi + 2, N_Z)

            bar = pltpu.get_barrier_semaphore()
            pl.semaphore_signal(bar, inc=1, device_id=(xi, yi, p0),
                                device_id_type=pl.DeviceIdType.MESH)
            pl.semaphore_signal(bar, inc=1, device_id=(xi, yi, p1),
                                device_id_type=pl.DeviceIdType.MESH)
            pl.semaphore_wait(bar, 2)

            r0 = pltpu.make_async_remote_copy(
                src_ref=acc_ref, dst_ref=c0_ref,
                send_sem=send_sems.at[0], recv_sem=recv_sems.at[0],
                device_id=(xi, yi, p0),
                device_id_type=pl.DeviceIdType.MESH,
            )
            r0.start()
            r0.wait()
            acc_ref[:, :] = acc_ref[:, :] + c0_ref[:, :]

            r1 = pltpu.make_async_remote_copy(
                src_ref=acc_ref, dst_ref=c1_ref,
                send_sem=send_sems.at[1], recv_sem=recv_sems.at[1],
                device_id=(xi, yi, p1),
                device_id_type=pl.DeviceIdType.MESH,
            )
            r1.start()
            r1.wait()
            out_ref[:, :] = acc_ref[:, :] + c1_ref[:, :]

    acc = pl.pallas_call(
        body,
        grid=(B,),
        in_specs=[
            pl.BlockSpec((1, HD, H), lambda b: (b, 0, 0)),
            pl.BlockSpec((1, Kloc, HD), lambda b: (b, 0, 0)),
            pl.BlockSpec((1, Kloc, HD), lambda b: (b, 0, 0)),
            pl.BlockSpec((H, HD), lambda b: (0, 0)),
        ],
        out_specs=pl.BlockSpec((B, LW), lambda b: (0, 0)),
        out_shape=jax.ShapeDtypeStruct((B, LW), jnp.float32),
        scratch_shapes=[
            pltpu.VMEM((B, LW), jnp.float32),
            pltpu.VMEM((B, LW), jnp.float32),
            pltpu.VMEM((B, LW), jnp.float32),
            pltpu.SemaphoreType.DMA((2,)),
            pltpu.SemaphoreType.DMA((2,)),
        ],
        compiler_params=pltpu.CompilerParams(collective_id=0),
    )(QB, K2, V2, M)

    o = acc[:, :HD].reshape(B, 1, H, D)
    lsum = acc[:, HD:].reshape(B, 1, H, 1)
    return o / lsum


# baseline (device time: 184819 ns/iter reference)
import jax
import jax.numpy as jnp
from jax import lax
from jax.experimental import pallas as pl
from jax.experimental.pallas import tpu as pltpu

N_Z = 4
SCALE = 64 ** -0.5


def kernel(Q, K, V):
    B, Sq, H, D = Q.shape
    _, Kloc, _, _ = K.shape
    HD = H * D
    LW = HD + 128

    K2 = K.reshape(B, Kloc, HD)
    V2 = V.reshape(B, Kloc, HD)

    eye = jnp.eye(H, dtype=Q.dtype)
    QB = (Q[:, 0, :, :, None] * eye[:, None, :]).reshape(B, HD, H)

    M = (jnp.arange(HD)[None, :] // D == jnp.arange(H)[:, None]).astype(
        jnp.float32
    )

    def body(qb_ref, k_ref, v_ref, m_ref, out_ref, acc_ref, c0_ref, c1_ref,
             send_sems, recv_sems):
        b = pl.program_id(0)

        s = lax.dot_general(
            k_ref[0], qb_ref[0],
            (((1,), (0,)), ((), ())),
            preferred_element_type=jnp.float32,
        )
        p = jnp.exp(s * SCALE)
        l = jnp.sum(p, axis=0, keepdims=True)
        o2 = lax.dot_general(
            p, v_ref[0],
            (((0,), (0,)), ((), ())),
            preferred_element_type=jnp.float32,
        )
        o_flat = jnp.sum(o2 * m_ref[:, :], axis=0, keepdims=True)

        row = jnp.concatenate(
            [o_flat, l, jnp.zeros((1, LW - HD - H), jnp.float32)], axis=1
        )
        acc_ref[pl.ds(b, 1), :] = row

        @pl.when(b == B - 1)
        def _comm():
            xi = lax.axis_index("x")
            yi = lax.axis_index("y")
            zi = lax.axis_index("z")
            p0 = zi + 1 - 2 * lax.rem(zi, 2)
            p1 = lax.rem(zi + 2, N_Z)

            bar = pltpu.get_barrier_semaphore()
            pl.semaphore_signal(bar, inc=1, device_id=(xi, yi, p0),
                                device_id_type=pl.DeviceIdType.MESH)
            pl.semaphore_signal(bar, inc=1, device_id=(xi, yi, p1),
                                device_id_type=pl.DeviceIdType.MESH)
            pl.semaphore_wait(bar, 2)

            r0 = pltpu.make_async_remote_copy(
                src_ref=acc_ref, dst_ref=c0_ref,
                send_sem=send_sems.at[0], recv_sem=recv_sems.at[0],
                device_id=(xi, yi, p0),
                device_id_type=pl.DeviceIdType.MESH,
            )
            r0.start()
            r0.wait()
            acc_ref[:, :] = acc_ref[:, :] + c0_ref[:, :]

            r1 = pltpu.make_async_remote_copy(
                src_ref=acc_ref, dst_ref=c1_ref,
                send_sem=send_sems.at[1], recv_sem=recv_sems.at[1],
                device_id=(xi, yi, p1),
                device_id_type=pl.DeviceIdType.MESH,
            )
            r1.start()
            r1.wait()
            out_ref[:, :] = acc_ref[:, :] + c1_ref[:, :]

    acc = pl.pallas_call(
        body,
        grid=(B,),
        in_specs=[
            pl.BlockSpec((1, HD, H), lambda b: (b, 0, 0)),
            pl.BlockSpec((1, Kloc, HD), lambda b: (b, 0, 0)),
            pl.BlockSpec((1, Kloc, HD), lambda b: (b, 0, 0)),
            pl.BlockSpec((H, HD), lambda b: (0, 0)),
        ],
        out_specs=pl.BlockSpec((B, LW), lambda b: (0, 0)),
        out_shape=jax.ShapeDtypeStruct((B, LW), jnp.float32),
        scratch_shapes=[
            pltpu.VMEM((B, LW), jnp.float32),
            pltpu.VMEM((B, LW), jnp.float32),
            pltpu.VMEM((B, LW), jnp.float32),
            pltpu.SemaphoreType.DMA((2,)),
            pltpu.SemaphoreType.DMA((2,)),
        ],
        compiler_params=pltpu.CompilerParams(collective_id=0),
    )(QB, K2, V2, M)

    o = acc[:, :HD].reshape(B, 1, H, D)
    lsum = acc[:, HD:HD + H].reshape(B, 1, H, 1)
    return o / lsum


# device time: 183502 ns/iter; 1.0072x vs baseline; 1.0072x over previous
import jax
import jax.numpy as jnp
from jax import lax
from jax.experimental import pallas as pl
from jax.experimental.pallas import tpu as pltpu

N_Z = 4
SCALE = 64 ** -0.5


def kernel(Q, K, V):
    B, Sq, H, D = Q.shape
    _, Kloc, _, _ = K.shape
    HD = H * D
    LW = HD + 128

    K2 = K.reshape(B, Kloc, HD)
    V2 = V.reshape(B, Kloc, HD)

    eye = jnp.eye(H, dtype=Q.dtype)
    QB = (Q[:, 0, :, :, None] * eye[:, None, :]).reshape(B, HD, H)

    M = (jnp.arange(HD)[None, :] // D == jnp.arange(H)[:, None]).astype(
        jnp.float32
    )

    def body(qb_ref, k_ref, v_ref, m_ref, out_ref, acc_ref, c0_ref, c1_ref,
             send_sems, recv_sems):
        b = pl.program_id(0)

        s = lax.dot_general(
            k_ref[0].astype(jnp.bfloat16), qb_ref[0].astype(jnp.bfloat16),
            (((1,), (0,)), ((), ())),
            preferred_element_type=jnp.float32,
        )
        p = jnp.exp(s * SCALE)
        l = jnp.sum(p, axis=0, keepdims=True)
        o2 = lax.dot_general(
            p.astype(jnp.bfloat16), v_ref[0].astype(jnp.bfloat16),
            (((0,), (0,)), ((), ())),
            preferred_element_type=jnp.float32,
        )
        o_flat = jnp.sum(o2 * m_ref[:, :], axis=0, keepdims=True)

        row = jnp.concatenate(
            [o_flat, l, jnp.zeros((1, LW - HD - H), jnp.float32)], axis=1
        )
        acc_ref[pl.ds(b, 1), :] = row

        @pl.when(b == B - 1)
        def _comm():
            xi = lax.axis_index("x")
            yi = lax.axis_index("y")
            zi = lax.axis_index("z")
            p0 = zi + 1 - 2 * lax.rem(zi, 2)
            p1 = lax.rem(zi + 2, N_Z)

            bar = pltpu.get_barrier_semaphore()
            pl.semaphore_signal(bar, inc=1, device_id=(xi, yi, p0),
                                device_id_type=pl.DeviceIdType.MESH)
            pl.semaphore_signal(bar, inc=1, device_id=(xi, yi, p1),
                                device_id_type=pl.DeviceIdType.MESH)
            pl.semaphore_wait(bar, 2)

            r0 = pltpu.make_async_remote_copy(
                src_ref=acc_ref, dst_ref=c0_ref,
                send_sem=send_sems.at[0], recv_sem=recv_sems.at[0],
                device_id=(xi, yi, p0),
                device_id_type=pl.DeviceIdType.MESH,
            )
            r0.start()
            r0.wait()
            acc_ref[:, :] = acc_ref[:, :] + c0_ref[:, :]

            r1 = pltpu.make_async_remote_copy(
                src_ref=acc_ref, dst_ref=c1_ref,
                send_sem=send_sems.at[1], recv_sem=recv_sems.at[1],
                device_id=(xi, yi, p1),
                device_id_type=pl.DeviceIdType.MESH,
            )
            r1.start()
            r1.wait()
            out_ref[:, :] = acc_ref[:, :] + c1_ref[:, :]

    acc = pl.pallas_call(
        body,
        grid=(B,),
        in_specs=[
            pl.BlockSpec((1, HD, H), lambda b: (b, 0, 0)),
            pl.BlockSpec((1, Kloc, HD), lambda b: (b, 0, 0)),
            pl.BlockSpec((1, Kloc, HD), lambda b: (b, 0, 0)),
            pl.BlockSpec((H, HD), lambda b: (0, 0)),
        ],
        out_specs=pl.BlockSpec((B, LW), lambda b: (0, 0)),
        out_shape=jax.ShapeDtypeStruct((B, LW), jnp.float32),
        scratch_shapes=[
            pltpu.VMEM((B, LW), jnp.float32),
            pltpu.VMEM((B, LW), jnp.float32),
            pltpu.VMEM((B, LW), jnp.float32),
            pltpu.SemaphoreType.DMA((2,)),
            pltpu.SemaphoreType.DMA((2,)),
        ],
        compiler_params=pltpu.CompilerParams(collective_id=0),
    )(QB, K2, V2, M)

    o = acc[:, :HD].reshape(B, 1, H, D)
    lsum = acc[:, HD:HD + H].reshape(B, 1, H, 1)
    return o / lsum


# device time: 54654 ns/iter; 3.3816x vs baseline; 3.3575x over previous
import jax
import jax.numpy as jnp
from jax import lax
from jax.experimental import pallas as pl
from jax.experimental.pallas import tpu as pltpu

N_Z = 4
SCALE = 64 ** -0.5


def kernel(Q, K, V):
    B, Sq, H, D = Q.shape
    _, Kloc, _, _ = K.shape
    HD = H * D
    LW = HD + 128
    HB = B // 2

    KT = K.transpose(0, 2, 3, 1)
    VT = V.transpose(0, 2, 3, 1)

    eye = jnp.eye(H, dtype=Q.dtype)
    QBD = (Q[:, 0][:, None, :, :] * eye[None, :, :, None]).reshape(B, H, HD)
    M2 = (jnp.arange(HD)[None, :] // D == jnp.arange(H)[:, None]).astype(
        jnp.float32
    )

    def body(qbd_ref, kt_ref, vt_ref, m_ref, out_ref, acc_ref, ca_ref,
             cb_ref, send_sems, recv_sems):
        b = pl.program_id(0)

        kt2 = kt_ref[0].reshape(HD, Kloc)
        vt2 = vt_ref[0].reshape(HD, Kloc)
        s = lax.dot_general(
            qbd_ref[0], kt2,
            (((1,), (0,)), ((), ())),
            preferred_element_type=jnp.float32,
        )
        p = jnp.exp(s * SCALE)
        lrow = lax.dot_general(
            jnp.ones((1, Kloc), jnp.float32), p,
            (((1,), (1,)), ((), ())),
            preferred_element_type=jnp.float32,
        )
        o_all = lax.dot_general(
            p, vt2,
            (((1,), (1,)), ((), ())),
            preferred_element_type=jnp.float32,
        )
        o_flat = jnp.sum(o_all * m_ref[:, :], axis=0, keepdims=True)
        row = jnp.concatenate(
            [o_flat, lrow, jnp.zeros((1, LW - HD - H), jnp.float32)], axis=1
        )
        acc_ref[pl.ds(b, 1), :] = row

        xi = lax.axis_index("x")
        yi = lax.axis_index("y")
        zi = lax.axis_index("z")

        def mk(j, src, dst, c):
            zp = lax.rem(zi + j + 1, N_Z)
            return pltpu.make_async_remote_copy(
                src_ref=src, dst_ref=dst,
                send_sem=send_sems.at[c, j], recv_sem=recv_sems.at[c, j],
                device_id=(xi, yi, zp),
                device_id_type=pl.DeviceIdType.MESH,
            )

        @pl.when(b == HB - 1)
        def _send0():
            bar = pltpu.get_barrier_semaphore()
            for j in range(N_Z - 1):
                zp = lax.rem(zi + j + 1, N_Z)
                pl.semaphore_signal(bar, inc=1, device_id=(xi, yi, zp),
                                    device_id_type=pl.DeviceIdType.MESH)
            pl.semaphore_wait(bar, N_Z - 1)
            for j in range(N_Z - 1):
                mk(j, acc_ref.at[pl.ds(0, HB)], ca_ref.at[j], 0).start()

        @pl.when(b == B - 1)
        def _finish():
            for j in range(N_Z - 1):
                mk(j, acc_ref.at[pl.ds(HB, HB)], cb_ref.at[j], 1).start()
            for j in range(N_Z - 1):
                mk(j, acc_ref.at[pl.ds(0, HB)], ca_ref.at[j], 0).wait()
            out_ref[pl.ds(0, HB), :] = (
                acc_ref[0:HB, :] + ca_ref[0] + ca_ref[1] + ca_ref[2]
            )
            for j in range(N_Z - 1):
                mk(j, acc_ref.at[pl.ds(HB, HB)], cb_ref.at[j], 1).wait()
            out_ref[pl.ds(HB, HB), :] = (
                acc_ref[HB:B, :] + cb_ref[0] + cb_ref[1] + cb_ref[2]
            )

    acc = pl.pallas_call(
        body,
        grid=(B,),
        in_specs=[
            pl.BlockSpec((1, H, HD), lambda b: (b, 0, 0)),
            pl.BlockSpec((1, H, D, Kloc), lambda b: (b, 0, 0, 0)),
            pl.BlockSpec((1, H, D, Kloc), lambda b: (b, 0, 0, 0)),
            pl.BlockSpec((H, HD), lambda b: (0, 0)),
        ],
        out_specs=pl.BlockSpec((B, LW), lambda b: (0, 0)),
        out_shape=jax.ShapeDtypeStruct((B, LW), jnp.float32),
        scratch_shapes=[
            pltpu.VMEM((B, LW), jnp.float32),
            pltpu.VMEM((N_Z - 1, HB, LW), jnp.float32),
            pltpu.VMEM((N_Z - 1, HB, LW), jnp.float32),
            pltpu.SemaphoreType.DMA((2, N_Z - 1)),
            pltpu.SemaphoreType.DMA((2, N_Z - 1)),
        ],
        compiler_params=pltpu.CompilerParams(
            collective_id=0, vmem_limit_bytes=96 * 1024 * 1024
        ),
    )(QBD, KT, VT, M2)

    o = acc[:, :HD].reshape(B, 1, H, D)
    lsum = acc[:, HD:HD + H].reshape(B, 1, H, 1)
    return o / lsum


# device time: 21820 ns/iter; 8.4702x vs baseline; 2.5048x over previous
import jax
import jax.numpy as jnp
from jax import lax
from jax.experimental import pallas as pl
from jax.experimental.pallas import tpu as pltpu

N_Z = 4
N_G = 8
SCALE = 64 ** -0.5


def kernel(Q, K, V):
    B, Sq, H, D = Q.shape
    _, Kloc, _, _ = K.shape
    HD = H * D
    LW = HD + 128
    NB = B // N_G

    KT = K.transpose(0, 2, 3, 1)
    VT = V.transpose(0, 2, 3, 1)
    QH = Q[:, 0]
    M2 = (jnp.arange(HD)[None, :] // D == jnp.arange(H)[:, None]).astype(
        jnp.float32
    )
    g0 = (lax.axis_index("x") * 4 + lax.axis_index("y")) * NB
    sidx = jnp.full((1,), g0, jnp.int32)

    def body(s_ref, q_ref, kt_ref, vt_ref, m_ref, out_ref, red_ref,
             norm_ref, cz0_ref, cz1_ref, send_z, recv_z, send_g, recv_g):
        i = pl.program_id(0)
        rb = s_ref[0]

        kt2 = kt_ref[0].reshape(HD, Kloc)
        vt2 = vt_ref[0].reshape(HD, Kloc)
        qbd = m_ref[:, :] * q_ref[0].reshape(1, HD)
        s = lax.dot_general(
            qbd, kt2,
            (((1,), (0,)), ((), ())),
            preferred_element_type=jnp.float32,
        )
        p = jnp.exp(s * SCALE)
        lrow = lax.dot_general(
            jnp.ones((1, Kloc), jnp.float32), p,
            (((1,), (1,)), ((), ())),
            preferred_element_type=jnp.float32,
        )
        o_all = lax.dot_general(
            p, vt2,
            (((1,), (1,)), ((), ())),
            preferred_element_type=jnp.float32,
        )
        o_flat = jnp.sum(o_all * m_ref[:, :], axis=0, keepdims=True)
        row = jnp.concatenate(
            [o_flat, lrow, jnp.zeros((1, LW - HD - H), jnp.float32)], axis=1
        )
        red_ref[pl.ds(i, 1), :] = row

        xi = lax.axis_index("x")
        yi = lax.axis_index("y")
        zi = lax.axis_index("z")
        g = xi * 4 + yi

        def mkz(r, j, cz):
            zp = lax.rem(zi + j + 1, N_Z)
            return pltpu.make_async_remote_copy(
                src_ref=red_ref.at[pl.ds(r, 1)],
                dst_ref=cz.at[pl.ds(j, 1)],
                send_sem=send_z.at[r, j], recv_sem=recv_z.at[r, j],
                device_id=(xi, yi, zp),
                device_id_type=pl.DeviceIdType.MESH,
            )

        def mkg(r, j):
            gp = lax.rem(g + j + 1, N_G)
            return pltpu.make_async_remote_copy(
                src_ref=norm_ref.at[pl.ds(r, 1)],
                dst_ref=out_ref.at[pl.ds(rb + r, 1)],
                send_sem=send_g.at[r, j], recv_sem=recv_g.at[r, j],
                device_id=(gp // 4, lax.rem(gp, 4), zi),
                device_id_type=pl.DeviceIdType.MESH,
            )

        def reduce_norm_bcast(r, cz):
            red = (red_ref[r:r + 1, :] + cz[0:1, :]
                   + cz[1:2, :] + cz[2:3, :])
            div = lax.dot_general(
                red[:, HD:HD + H], m_ref[:, :],
                (((1,), (0,)), ((), ())),
                preferred_element_type=jnp.float32,
            )
            norm_ref[pl.ds(r, 1), :] = red[:, :HD] / div
            out_ref[pl.ds(rb + r, 1), :] = norm_ref[r:r + 1, :]
            for j in range(N_G - 1):
                mkg(r, j).start()

        @pl.when(i == 0)
        def _send_row0():
            bar = pltpu.get_barrier_semaphore()
            for j in range(N_Z - 1):
                zp = lax.rem(zi + j + 1, N_Z)
                pl.semaphore_signal(bar, inc=1, device_id=(xi, yi, zp),
                                    device_id_type=pl.DeviceIdType.MESH)
            for j in range(N_G - 1):
                gp = lax.rem(g + j + 1, N_G)
                pl.semaphore_signal(
                    bar, inc=1, device_id=(gp // 4, lax.rem(gp, 4), zi),
                    device_id_type=pl.DeviceIdType.MESH)
            pl.semaphore_wait(bar, N_Z - 1 + N_G - 1)
            for j in range(N_Z - 1):
                mkz(0, j, cz0_ref).start()

        @pl.when(i == NB - 1)
        def _finish():
            for j in range(N_Z - 1):
                mkz(1, j, cz1_ref).start()
            for j in range(N_Z - 1):
                mkz(0, j, cz0_ref).wait()
            reduce_norm_bcast(0, cz0_ref)
            for j in range(N_Z - 1):
                mkz(1, j, cz1_ref).wait()
            reduce_norm_bcast(1, cz1_ref)
            for r in range(NB):
                for j in range(N_G - 1):
                    mkg(r, j).wait()

    grid_spec = pltpu.PrefetchScalarGridSpec(
        num_scalar_prefetch=1,
        grid=(NB,),
        in_specs=[
            pl.BlockSpec((1, H, D), lambda i, s: (s[0] + i, 0, 0)),
            pl.BlockSpec((1, H, D, Kloc), lambda i, s: (s[0] + i, 0, 0, 0)),
            pl.BlockSpec((1, H, D, Kloc), lambda i, s: (s[0] + i, 0, 0, 0)),
            pl.BlockSpec((H, HD), lambda i, s: (0, 0)),
        ],
        out_specs=pl.BlockSpec((B, HD), lambda i, s: (0, 0)),
        scratch_shapes=[
            pltpu.VMEM((NB, LW), jnp.float32),
            pltpu.VMEM((NB, HD), jnp.float32),
            pltpu.VMEM((N_Z - 1, LW), jnp.float32),
            pltpu.VMEM((N_Z - 1, LW), jnp.float32),
            pltpu.SemaphoreType.DMA((NB, N_Z - 1)),
            pltpu.SemaphoreType.DMA((NB, N_Z - 1)),
            pltpu.SemaphoreType.DMA((NB, N_G - 1)),
            pltpu.SemaphoreType.DMA((NB, N_G - 1)),
        ],
    )
    acc = pl.pallas_call(
        body,
        grid_spec=grid_spec,
        out_shape=jax.ShapeDtypeStruct((B, HD), jnp.float32),
        compiler_params=pltpu.CompilerParams(
            collective_id=0, vmem_limit_bytes=96 * 1024 * 1024
        ),
    )(sidx, QH, KT, VT, M2)

    return acc.reshape(B, 1, H, D)
